# Initial kernel scaffold; baseline (speedup 1.0000x reference)
import jax
import jax.numpy as jnp
from jax.experimental import pallas as pl


def kernel(conv1_w, conv1_b, conv2_w, conv2_b, conv3_w, conv3_b, fc1_w, fc1_b, fc2_w, fc2_b, fc3_w, fc3_b, x_nchw):
    raise NotImplementedError("write your pallas kernel here")


# speedup vs baseline: 1.1534x; 1.1534x over previous
import functools

import jax
import jax.numpy as jnp
from jax.experimental import pallas as pl
from jax.experimental.pallas import tpu as pltpu

_NEG_SLOPE = 0.01


def _leaky(z):
    return jnp.where(z > 0, z, _NEG_SLOPE * z)


def _c1_body(p_ref, w_ref, b_ref, o_ref):
    acc = jnp.dot(p_ref[0], w_ref[...], preferred_element_type=jnp.float32)
    z = _leaky(acc + b_ref[...])
    o_ref[0] = z.astype(jnp.bfloat16)


def _conv1(patches, w, b):
    n, m, k = patches.shape
    cout = w.shape[1]
    return pl.pallas_call(
        _c1_body,
        out_shape=jax.ShapeDtypeStruct((n, m, cout), jnp.bfloat16),
        grid=(n,),
        in_specs=[
            pl.BlockSpec((1, m, k), lambda i: (i, 0, 0)),
            pl.BlockSpec((k, cout), lambda i: (0, 0)),
            pl.BlockSpec((1, cout), lambda i: (0, 0)),
        ],
        out_specs=pl.BlockSpec((1, m, cout), lambda i: (i, 0, 0)),
        compiler_params=pltpu.CompilerParams(
            dimension_semantics=("parallel",)),
    )(patches, w, b)


def _conv_body(x_ref, w_ref, b_ref, o_ref, *, offsets, m):
    x = x_ref[0]
    acc = jnp.dot(x[offsets[0]:offsets[0] + m, :], w_ref[0],
                  preferred_element_type=jnp.float32)
    for t in range(1, len(offsets)):
        off = offsets[t]
        acc = acc + jnp.dot(x[off:off + m, :], w_ref[t],
                            preferred_element_type=jnp.float32)
    z = _leaky(acc + b_ref[...])
    o_ref[0] = z.astype(jnp.bfloat16)


def _conv_taps(x, w, b, *, kh, kw, wstride, m):
    n, rows, cin = x.shape
    taps, _, cout = w.shape
    assert taps == kh * kw
    offsets = tuple(dh * wstride + dw for dh in range(kh) for dw in range(kw))
    return pl.pallas_call(
        functools.partial(_conv_body, offsets=offsets, m=m),
        out_shape=jax.ShapeDtypeStruct((n, m, cout), jnp.bfloat16),
        grid=(n,),
        in_specs=[
            pl.BlockSpec((1, rows, cin), lambda i: (i, 0, 0)),
            pl.BlockSpec((taps, cin, cout), lambda i: (0, 0, 0)),
            pl.BlockSpec((1, cout), lambda i: (0, 0)),
        ],
        out_specs=pl.BlockSpec((1, m, cout), lambda i: (i, 0, 0)),
        compiler_params=pltpu.CompilerParams(
            dimension_semantics=("parallel",)),
    )(x, w, b)


def _pool_body(ee_ref, eo_ref, oe_ref, oo_ref, o_ref, *, oh, ow, pad_h, pad_w):
    m = ee_ref[:, 0:oh, 0:ow, :]
    m = jnp.maximum(m, ee_ref[:, 0:oh, 1:ow + 1, :])
    m = jnp.maximum(m, ee_ref[:, 1:oh + 1, 0:ow, :])
    m = jnp.maximum(m, ee_ref[:, 1:oh + 1, 1:ow + 1, :])
    m = jnp.maximum(m, eo_ref[:, 0:oh, 0:ow, :])
    m = jnp.maximum(m, eo_ref[:, 1:oh + 1, 0:ow, :])
    m = jnp.maximum(m, oe_ref[:, 0:oh, 0:ow, :])
    m = jnp.maximum(m, oe_ref[:, 0:oh, 1:ow + 1, :])
    m = jnp.maximum(m, oo_ref[:, 0:oh, 0:ow, :])
    if pad_w:
        zw = jnp.zeros(m.shape[:2] + (pad_w, m.shape[3]), m.dtype)
        m = jnp.concatenate([m, zw], axis=2)
    if pad_h:
        zh = jnp.zeros((m.shape[0], pad_h) + m.shape[2:], m.dtype)
        m = jnp.concatenate([m, zh], axis=1)
    o_ref[...] = m


def _maxpool(x, *, pad_h=0, pad_w=0):
    n, h, w, c = x.shape
    oh, ow = h // 2, w // 2
    neg = jnp.finfo(x.dtype).min
    xp = jnp.pad(x, ((0, 0), (1, 1), (1, 1), (0, 0)), constant_values=neg)
    ee = xp[:, 0::2, 0::2, :]
    eo = xp[:, 0::2, 1::2, :]
    oe = xp[:, 1::2, 0::2, :]
    oo = xp[:, 1::2, 1::2, :]
    hp, wp = ee.shape[1], ee.shape[2]
    plane = pl.BlockSpec((1, hp, wp, c), lambda i: (i, 0, 0, 0))
    return pl.pallas_call(
        functools.partial(_pool_body, oh=oh, ow=ow, pad_h=pad_h, pad_w=pad_w),
        out_shape=jax.ShapeDtypeStruct((n, oh + pad_h, ow + pad_w, c),
                                       x.dtype),
        grid=(n,),
        in_specs=[plane] * 4,
        out_specs=pl.BlockSpec((1, oh + pad_h, ow + pad_w, c),
                               lambda i: (i, 0, 0, 0)),
        compiler_params=pltpu.CompilerParams(
            dimension_semantics=("parallel",)),
    )(ee, eo, oe, oo)


def _fc_body(a_ref, w1_ref, b1_ref, w2_ref, b2_ref, w3_ref, b3_ref, o_ref,
             *, valid_cols):
    h1 = _leaky(jnp.dot(a_ref[...], w1_ref[...],
                        preferred_element_type=jnp.float32) + b1_ref[...])
    h2 = _leaky(jnp.dot(h1.astype(jnp.bfloat16), w2_ref[...],
                        preferred_element_type=jnp.float32) + b2_ref[...])
    z = jnp.dot(h2.astype(jnp.bfloat16), w3_ref[...],
                preferred_element_type=jnp.float32) + b3_ref[...]
    col = jax.lax.broadcasted_iota(jnp.int32, z.shape, 1)
    valid = col < valid_cols
    zm = jnp.where(valid, z, -jnp.inf)
    mx = jnp.max(zm, axis=-1, keepdims=True)
    e = jnp.where(valid, jnp.exp(z - mx), 0.0)
    lse = jnp.log(jnp.sum(e, axis=-1, keepdims=True)) + mx
    o_ref[...] = z - lse


def _fc_fused(a, w1, b1, w2, b2, w3, b3, *, valid_cols):
    m, k1 = a.shape
    n1 = w1.shape[1]
    n2 = w2.shape[1]
    n3 = w3.shape[1]
    full = lambda shape: pl.BlockSpec(shape, lambda: tuple(0 for _ in shape))
    return pl.pallas_call(
        functools.partial(_fc_body, valid_cols=valid_cols),
        out_shape=jax.ShapeDtypeStruct((m, n3), jnp.float32),
        in_specs=[
            full((m, k1)),
            full((k1, n1)), full((1, n1)),
            full((n1, n2)), full((1, n2)),
            full((n2, n3)), full((1, n3)),
        ],
        out_specs=full((m, n3)),
        compiler_params=pltpu.CompilerParams(
            vmem_limit_bytes=100 * 1024 * 1024),
    )(a, w1, b1, w2, b2, w3, b3)


def kernel(conv1_w, conv1_b, conv2_w, conv2_b, conv3_w, conv3_b,
           fc1_w, fc1_b, fc2_w, fc2_b, fc3_w, fc3_b, x_nchw):
    bf = jnp.bfloat16
    n = x_nchw.shape[0]

    x = jnp.transpose(x_nchw.astype(bf), (0, 2, 3, 1))
    cols = []
    for dh in range(7):
        for dw in range(7):
            cols.append(x[:, dh:dh + 221:4, dw:dw + 221:4, :])
    patches = jnp.concatenate(cols, axis=-1)
    patches = patches.reshape(n, 56 * 56, 147)

    y1 = _conv1(patches, conv1_w.astype(bf), conv1_b)
    y1 = y1.reshape(n, 56, 56, 128)
    p1 = _maxpool(y1, pad_h=1, pad_w=4)

    y2 = _conv_taps(p1.reshape(n, 29 * 32, 128),
                    conv2_w.astype(bf), conv2_b,
                    kh=5, kw=5, wstride=32, m=768)
    y2 = y2.reshape(n, 24, 32, 256)[:, :, :24, :]
    p2 = _maxpool(y2, pad_h=1, pad_w=4)

    y3 = _conv_taps(p2.reshape(n, 13 * 16, 256),
                    conv3_w.astype(bf), conv3_b,
                    kh=3, kw=3, wstride=16, m=160)
    y3 = y3.reshape(n, 10, 16, 384)[:, :, :10, :]
    p3 = _maxpool(y3)

    a = p3.reshape(n, 9600)
    out = _fc_fused(a, fc1_w.astype(bf), fc1_b, fc2_w.astype(bf), fc2_b,
                    fc3_w.astype(bf), fc3_b, valid_cols=6)
    return out[:, :6]


# speedup vs baseline: 2.1998x; 1.9073x over previous
import functools

import jax
import jax.numpy as jnp
from jax.experimental import pallas as pl
from jax.experimental.pallas import tpu as pltpu

_NEG_SLOPE = 0.01
_NEG = -1e30


def _leaky(z):
    return jnp.where(z > 0, z, _NEG_SLOPE * z)


def _pool_sel(p_rows, q_rows, wo, wi, valid_h, valid_w):
    p = jnp.arange(p_rows)[:, None]
    q = jnp.arange(q_rows)[None, :]
    i, j = p // wo, p % wo
    valid = (i < valid_h) & (j < valid_w)
    target = jnp.where(valid, 2 * wi * i + 2 * j, -1)
    return (q == target).astype(jnp.bfloat16)


def _pool_flat(z, *, ws, valid_w, masked_edges):
    m, c = z.shape
    r = jax.lax.broadcasted_iota(jnp.int32, (m, c), 0)
    w = r % ws
    if valid_w < ws:
        z = jnp.where(w < valid_w, z, _NEG)
    neg_row = jnp.full((1, c), _NEG, jnp.float32)
    sd = jnp.concatenate([neg_row, z[:-1]], axis=0)
    su = jnp.concatenate([z[1:], neg_row], axis=0)
    if masked_edges:
        sd = jnp.where(w > 0, sd, _NEG)
        su = jnp.where(w < ws - 1, su, _NEG)
    m1 = jnp.maximum(jnp.maximum(sd, su), z)
    neg_band = jnp.full((ws, c), _NEG, jnp.float32)
    md = jnp.concatenate([neg_band, m1[:-ws]], axis=0)
    mu = jnp.concatenate([m1[ws:], neg_band], axis=0)
    return jnp.maximum(jnp.maximum(md, mu), m1)


def _c1_body(p_ref, w_ref, b_ref, s_ref, o_ref):
    z = jnp.dot(p_ref[0], w_ref[...], preferred_element_type=jnp.float32)
    z = _leaky(z + b_ref[...])
    m2 = _pool_flat(z, ws=56, valid_w=56, masked_edges=True)
    o_ref[0] = jnp.dot(s_ref[...], m2.astype(jnp.bfloat16),
                       preferred_element_type=jnp.float32).astype(jnp.bfloat16)


def _conv1_pool(patches, w, b, sel):
    n, m, k = patches.shape
    cout = w.shape[1]
    pr = sel.shape[0]
    return pl.pallas_call(
        _c1_body,
        out_shape=jax.ShapeDtypeStruct((n, pr, cout), jnp.bfloat16),
        grid=(n,),
        in_specs=[
            pl.BlockSpec((1, m, k), lambda i: (i, 0, 0)),
            pl.BlockSpec((k, cout), lambda i: (0, 0)),
            pl.BlockSpec((1, cout), lambda i: (0, 0)),
            pl.BlockSpec((pr, m), lambda i: (0, 0)),
        ],
        out_specs=pl.BlockSpec((1, pr, cout), lambda i: (i, 0, 0)),
        compiler_params=pltpu.CompilerParams(
            dimension_semantics=("parallel",)),
    )(patches, w, b, sel)


def _conv_body(x_ref, w_ref, b_ref, s_ref, o_ref, *, offsets, m, ws, valid_w):
    x = x_ref[0]
    acc = jnp.dot(x[offsets[0]:offsets[0] + m, :], w_ref[0],
                  preferred_element_type=jnp.float32)
    for t in range(1, len(offsets)):
        off = offsets[t]
        acc = acc + jnp.dot(x[off:off + m, :], w_ref[t],
                            preferred_element_type=jnp.float32)
    z = _leaky(acc + b_ref[...])
    m2 = _pool_flat(z, ws=ws, valid_w=valid_w, masked_edges=False)
    o_ref[0] = jnp.dot(s_ref[...], m2.astype(jnp.bfloat16),
                       preferred_element_type=jnp.float32).astype(jnp.bfloat16)


def _conv_pool(x, w, b, sel, *, kh, kw, ws, m, valid_w):
    n, rows, cin = x.shape
    taps, _, cout = w.shape
    assert taps == kh * kw
    pr = sel.shape[0]
    offsets = tuple(dh * ws + dw for dh in range(kh) for dw in range(kw))
    return pl.pallas_call(
        functools.partial(_conv_body, offsets=offsets, m=m, ws=ws,
                          valid_w=valid_w),
        out_shape=jax.ShapeDtypeStruct((n, pr, cout), jnp.bfloat16),
        grid=(n,),
        in_specs=[
            pl.BlockSpec((1, rows, cin), lambda i: (i, 0, 0)),
            pl.BlockSpec((taps, cin, cout), lambda i: (0, 0, 0)),
            pl.BlockSpec((1, cout), lambda i: (0, 0)),
            pl.BlockSpec((pr, m), lambda i: (0, 0)),
        ],
        out_specs=pl.BlockSpec((1, pr, cout), lambda i: (i, 0, 0)),
        compiler_params=pltpu.CompilerParams(
            dimension_semantics=("parallel",)),
    )(x, w, b, sel)


def _fc_body(a_ref, w1_ref, b1_ref, w2_ref, b2_ref, w3_ref, b3_ref, o_ref,
             *, valid_cols):
    h1 = _leaky(jnp.dot(a_ref[...], w1_ref[...],
                        preferred_element_type=jnp.float32) + b1_ref[...])
    h2 = _leaky(jnp.dot(h1.astype(jnp.bfloat16), w2_ref[...],
                        preferred_element_type=jnp.float32) + b2_ref[...])
    z = jnp.dot(h2.astype(jnp.bfloat16), w3_ref[...],
                preferred_element_type=jnp.float32) + b3_ref[...]
    col = jax.lax.broadcasted_iota(jnp.int32, z.shape, 1)
    valid = col < valid_cols
    zm = jnp.where(valid, z, -jnp.inf)
    mx = jnp.max(zm, axis=-1, keepdims=True)
    e = jnp.where(valid, jnp.exp(z - mx), 0.0)
    lse = jnp.log(jnp.sum(e, axis=-1, keepdims=True)) + mx
    o_ref[...] = z - lse


def _fc_fused(a, w1, b1, w2, b2, w3, b3, *, valid_cols):
    m, k1 = a.shape
    n1, n2, n3 = w1.shape[1], w2.shape[1], w3.shape[1]
    full = lambda shape: pl.BlockSpec(shape, lambda: tuple(0 for _ in shape))
    return pl.pallas_call(
        functools.partial(_fc_body, valid_cols=valid_cols),
        out_shape=jax.ShapeDtypeStruct((m, n3), jnp.float32),
        in_specs=[
            full((m, k1)),
            full((k1, n1)), full((1, n1)),
            full((n1, n2)), full((1, n2)),
            full((n2, n3)), full((1, n3)),
        ],
        out_specs=full((m, n3)),
        compiler_params=pltpu.CompilerParams(
            vmem_limit_bytes=100 * 1024 * 1024),
    )(a, w1, b1, w2, b2, w3, b3)


def kernel(conv1_w, conv1_b, conv2_w, conv2_b, conv3_w, conv3_b,
           fc1_w, fc1_b, fc2_w, fc2_b, fc3_w, fc3_b, x_nchw):
    bf = jnp.bfloat16
    n = x_nchw.shape[0]

    x = jnp.transpose(x_nchw.astype(bf), (0, 2, 3, 1))
    cols = []
    for dh in range(7):
        for dw in range(7):
            cols.append(x[:, dh:dh + 221:4, dw:dw + 221:4, :])
    patches = jnp.concatenate(cols, axis=-1)
    patches = patches.reshape(n, 56 * 56, 147)

    s1 = _pool_sel(928, 3136, 32, 56, 28, 28)
    s2 = _pool_sel(208, 768, 16, 32, 12, 12)
    s3 = _pool_sel(25, 160, 5, 16, 5, 5)

    p1 = _conv1_pool(patches, conv1_w.astype(bf), conv1_b, s1)
    p2 = _conv_pool(p1, conv2_w.astype(bf), conv2_b, s2,
                    kh=5, kw=5, ws=32, m=768, valid_w=24)
    p3 = _conv_pool(p2, conv3_w.astype(bf), conv3_b, s3,
                    kh=3, kw=3, ws=16, m=160, valid_w=10)

    a = p3.reshape(n, 9600)
    out = _fc_fused(a, fc1_w.astype(bf), fc1_b, fc2_w.astype(bf), fc2_b,
                    fc3_w.astype(bf), fc3_b, valid_cols=6)
    return out[:, :6]


# speedup vs baseline: 20.1368x; 9.1540x over previous
import functools

import jax
import jax.numpy as jnp
from jax.experimental import pallas as pl
from jax.experimental.pallas import tpu as pltpu

_NEG_SLOPE = 0.01
_NEG = -1e30


def _leaky(z):
    return jnp.where(z > 0, z, _NEG_SLOPE * z)


def _pool_sel(p_rows, q_rows, wo, wi, valid_h, valid_w):
    p = jnp.arange(p_rows)[:, None]
    q = jnp.arange(q_rows)[None, :]
    i, j = p // wo, p % wo
    valid = (i < valid_h) & (j < valid_w)
    target = jnp.where(valid, 2 * wi * i + 2 * j, -1)
    return (q == target).astype(jnp.bfloat16)


def _pool_flat(z, *, ws, valid_w, masked_edges):
    m, c = z.shape
    r = jax.lax.broadcasted_iota(jnp.int32, (m, c), 0)
    w = r % ws
    if valid_w < ws:
        z = jnp.where(w < valid_w, z, _NEG)
    neg_row = jnp.full((1, c), _NEG, jnp.float32)
    sd = jnp.concatenate([neg_row, z[:-1]], axis=0)
    su = jnp.concatenate([z[1:], neg_row], axis=0)
    if masked_edges:
        sd = jnp.where(w > 0, sd, _NEG)
        su = jnp.where(w < ws - 1, su, _NEG)
    m1 = jnp.maximum(jnp.maximum(sd, su), z)
    neg_band = jnp.full((ws, c), _NEG, jnp.float32)
    md = jnp.concatenate([neg_band, m1[:-ws]], axis=0)
    mu = jnp.concatenate([m1[ws:], neg_band], axis=0)
    return jnp.maximum(jnp.maximum(md, mu), m1)


def _c1_body(p_ref, w_ref, b_ref, s_ref, o_ref):
    z = jnp.dot(p_ref[0], w_ref[...], preferred_element_type=jnp.float32)
    z = _leaky(z + b_ref[...])
    m2 = _pool_flat(z, ws=56, valid_w=56, masked_edges=True)
    o_ref[0] = jnp.dot(s_ref[...], m2.astype(jnp.bfloat16),
                       preferred_element_type=jnp.float32).astype(jnp.bfloat16)


def _conv1_pool(patches, w, b, sel):
    n, m, k = patches.shape
    cout = w.shape[1]
    pr = sel.shape[0]
    return pl.pallas_call(
        _c1_body,
        out_shape=jax.ShapeDtypeStruct((n, pr, cout), jnp.bfloat16),
        grid=(n,),
        in_specs=[
            pl.BlockSpec((1, m, k), lambda i: (i, 0, 0)),
            pl.BlockSpec((k, cout), lambda i: (0, 0)),
            pl.BlockSpec((1, cout), lambda i: (0, 0)),
            pl.BlockSpec((pr, m), lambda i: (0, 0)),
        ],
        out_specs=pl.BlockSpec((1, pr, cout), lambda i: (i, 0, 0)),
        compiler_params=pltpu.CompilerParams(
            dimension_semantics=("parallel",)),
    )(patches, w, b, sel)


def _conv_body(x_ref, w_ref, b_ref, s_ref, o_ref, *, offsets, m, ws, valid_w):
    x = x_ref[0]
    acc = jnp.dot(x[offsets[0]:offsets[0] + m, :], w_ref[0],
                  preferred_element_type=jnp.float32)
    for t in range(1, len(offsets)):
        off = offsets[t]
        acc = acc + jnp.dot(x[off:off + m, :], w_ref[t],
                            preferred_element_type=jnp.float32)
    z = _leaky(acc + b_ref[...])
    m2 = _pool_flat(z, ws=ws, valid_w=valid_w, masked_edges=False)
    o_ref[0] = jnp.dot(s_ref[...], m2.astype(jnp.bfloat16),
                       preferred_element_type=jnp.float32).astype(jnp.bfloat16)


def _conv_pool(x, w, b, sel, *, kh, kw, ws, m, valid_w):
    n, rows, cin = x.shape
    taps, _, cout = w.shape
    assert taps == kh * kw
    pr = sel.shape[0]
    offsets = tuple(dh * ws + dw for dh in range(kh) for dw in range(kw))
    return pl.pallas_call(
        functools.partial(_conv_body, offsets=offsets, m=m, ws=ws,
                          valid_w=valid_w),
        out_shape=jax.ShapeDtypeStruct((n, pr, cout), jnp.bfloat16),
        grid=(n,),
        in_specs=[
            pl.BlockSpec((1, rows, cin), lambda i: (i, 0, 0)),
            pl.BlockSpec((taps, cin, cout), lambda i: (0, 0, 0)),
            pl.BlockSpec((1, cout), lambda i: (0, 0)),
            pl.BlockSpec((pr, m), lambda i: (0, 0)),
        ],
        out_specs=pl.BlockSpec((1, pr, cout), lambda i: (i, 0, 0)),
        compiler_params=pltpu.CompilerParams(
            dimension_semantics=("parallel",)),
    )(x, w, b, sel)


def _fc_body(a_ref, w1_ref, b1_ref, w2_ref, b2_ref, w3_ref, b3_ref, o_ref,
             *, valid_cols):
    h1 = _leaky(jnp.dot(a_ref[...], w1_ref[...],
                        preferred_element_type=jnp.float32) + b1_ref[...])
    h2 = _leaky(jnp.dot(h1.astype(jnp.bfloat16), w2_ref[...],
                        preferred_element_type=jnp.float32) + b2_ref[...])
    z = jnp.dot(h2.astype(jnp.bfloat16), w3_ref[...],
                preferred_element_type=jnp.float32) + b3_ref[...]
    col = jax.lax.broadcasted_iota(jnp.int32, z.shape, 1)
    valid = col < valid_cols
    zm = jnp.where(valid, z, -jnp.inf)
    mx = jnp.max(zm, axis=-1, keepdims=True)
    e = jnp.where(valid, jnp.exp(z - mx), 0.0)
    lse = jnp.log(jnp.sum(e, axis=-1, keepdims=True)) + mx
    o_ref[...] = z - lse


def _fc_fused(a, w1, b1, w2, b2, w3, b3, *, valid_cols):
    m, k1 = a.shape
    n1, n2, n3 = w1.shape[1], w2.shape[1], w3.shape[1]
    full = lambda shape: pl.BlockSpec(shape, lambda: tuple(0 for _ in shape))
    return pl.pallas_call(
        functools.partial(_fc_body, valid_cols=valid_cols),
        out_shape=jax.ShapeDtypeStruct((m, n3), jnp.float32),
        in_specs=[
            full((m, k1)),
            full((k1, n1)), full((1, n1)),
            full((n1, n2)), full((1, n2)),
            full((n2, n3)), full((1, n3)),
        ],
        out_specs=full((m, n3)),
        compiler_params=pltpu.CompilerParams(
            vmem_limit_bytes=100 * 1024 * 1024),
    )(a, w1, b1, w2, b2, w3, b3)


def kernel(conv1_w, conv1_b, conv2_w, conv2_b, conv3_w, conv3_b,
           fc1_w, fc1_b, fc2_w, fc2_b, fc3_w, fc3_b, x_nchw):
    bf = jnp.bfloat16
    n = x_nchw.shape[0]

    patches = jax.lax.conv_general_dilated_patches(
        x_nchw.astype(bf), filter_shape=(7, 7), window_strides=(4, 4),
        padding="VALID",
        dimension_numbers=("NCHW", "OIHW", "NHWC"))
    patches = patches.reshape(n, 56 * 56, 147)
    w1 = conv1_w.reshape(7, 7, 3, 128).transpose(2, 0, 1, 3).reshape(147, 128)

    s1 = _pool_sel(928, 3136, 32, 56, 28, 28)
    s2 = _pool_sel(208, 768, 16, 32, 12, 12)
    s3 = _pool_sel(25, 160, 5, 16, 5, 5)

    p1 = _conv1_pool(patches, w1.astype(bf), conv1_b, s1)
    p2 = _conv_pool(p1, conv2_w.astype(bf), conv2_b, s2,
                    kh=5, kw=5, ws=32, m=768, valid_w=24)
    p3 = _conv_pool(p2, conv3_w.astype(bf), conv3_b, s3,
                    kh=3, kw=3, ws=16, m=160, valid_w=10)

    a = p3.reshape(n, 9600)
    out = _fc_fused(a, fc1_w.astype(bf), fc1_b, fc2_w.astype(bf), fc2_b,
                    fc3_w.astype(bf), fc3_b, valid_cols=6)
    return out[:, :6]
